# baseline (speedup 1.0000x reference)
import functools

import jax
import jax.numpy as jnp
from jax import lax
from jax.experimental import pallas as pl
from jax.experimental.pallas import tpu as pltpu
from jax.experimental.pallas import tpu_sc as plsc

N = 320000
D = 128
M = 10000


FOLD = 16
KW = D * FOLD
SPLIT = 128000


def _matvec(t0, g, b, n_atoms, off_atoms, br):
    blk = br * FOLD
    ng = n_atoms // blk
    offb = off_atoms // blk

    def _mv_body(t_ref, g_ref, b_ref, o_ref):
        t = t_ref[...].reshape(br, KW)
        y = lax.dot_general(t, g_ref[...], (((1,), (0,)), ((), ())),
                            preferred_element_type=jnp.float32)
        o_ref[...] = y + b_ref[0]

    return pl.pallas_call(
        _mv_body,
        grid=(ng,),
        in_specs=[
            pl.BlockSpec((blk, D), lambda i: (i + offb, 0)),
            pl.BlockSpec((KW, FOLD), lambda i: (0, 0)),
            pl.BlockSpec(memory_space=pltpu.SMEM),
        ],
        out_specs=pl.BlockSpec((br, FOLD), lambda i: (i, 0)),
        out_shape=jax.ShapeDtypeStruct((n_atoms // FOLD, FOLD), jnp.float32),
    )(t0, g, b)


NC = 2
NS = 16
NW = NC * NS
MP = 10240
RED = MP // NS


def _gather16(x, idx):
    return lax.gather(
        x, idx[:, None],
        dimension_numbers=lax.GatherDimensionNumbers(
            offset_dims=(), collapsed_slice_dims=(0,), start_index_map=(0,)),
        slice_sizes=(1,),
        mode=lax.GatherScatterMode.PROMISE_IN_BOUNDS)


def _make_sc_body(chunk, idx_off):
    nvec = chunk // 16

    def _sc_body(y_hbm, idx_hbm, out_hbm, yv, iv, tbl, red, acc, shared):
        c = lax.axis_index("c")
        s = lax.axis_index("s")
        wid = c * NS + s
        base = wid * chunk
        pltpu.sync_copy(y_hbm.at[pl.ds(base, chunk)], yv)
        pltpu.sync_copy(idx_hbm.at[pl.ds(idx_off + base, chunk)], iv)

        zero = jnp.zeros((16,), jnp.float32)
        iota = lax.iota(jnp.int32, 16)
        last = iota == 15
        notlast = jnp.logical_not(last)

        def zbody(i, _):
            tbl[pl.ds(i * 16, 16)] = zero
            return 0

        lax.fori_loop(0, MP // 16, zbody, 0, unroll=8)

        def scat(yk, ik, ink):
            sk = plsc.cumsum(yk)
            e = ik != ink
            plsc.addupdate_scatter(tbl, [ik], sk,
                                   mask=jnp.logical_or(e, last))
            plsc.addupdate_scatter(tbl, [ink], -sk,
                                   mask=jnp.logical_and(e, notlast))

        def body(k, _):
            off = k * 16
            scat(yv[pl.ds(off, 16)], iv[pl.ds(off, 16)],
                 iv[pl.ds(off + 1, 16)])
            return 0

        lax.fori_loop(0, nvec - 1, body, 0, unroll=5)

        off = (nvec - 1) * 16
        ik = iv[pl.ds(off, 16)]
        scat(yv[pl.ds(off, 16)], ik,
             _gather16(ik, jnp.minimum(iota + 1, 15)))

        pltpu.sync_copy(tbl, shared.at[s])
        plsc.subcore_barrier()

        rbase = s * RED
        for t in range(NS):
            pltpu.sync_copy(shared.at[t, pl.ds(rbase, RED)], red.at[t])

        def rbody(i, _):
            off = i * 16
            v = red[0, pl.ds(off, 16)]
            for t in range(1, NS):
                v = v + red[t, pl.ds(off, 16)]
            acc[pl.ds(off, 16)] = v
            return 0

        lax.fori_loop(0, RED // 16, rbody, 0, unroll=2)
        pltpu.sync_copy(acc, out_hbm.at[c, pl.ds(rbase, RED)])

    return _sc_body


def _make_segsum(n_atoms, idx_off):
    chunk = n_atoms // NW
    return functools.partial(
        pl.kernel,
        mesh=plsc.VectorSubcoreMesh(core_axis_name="c", subcore_axis_name="s"),
        out_type=jax.ShapeDtypeStruct((NC, MP), jnp.float32),
        compiler_params=pltpu.CompilerParams(needs_layout_passes=False),
        scratch_types=[
            pltpu.VMEM((chunk,), jnp.float32),
            pltpu.VMEM((chunk,), jnp.int32),
            pltpu.VMEM((MP,), jnp.float32),
            pltpu.VMEM((NS, RED), jnp.float32),
            pltpu.VMEM((RED,), jnp.float32),
            pltpu.VMEM_SHARED((NS, MP), jnp.float32),
        ],
    )(_make_sc_body(chunk, idx_off))


_segsum_a = _make_segsum(SPLIT, 0)
_segsum_b = _make_segsum(N - SPLIT, SPLIT)


MR = MP // 128


def _comb_body(pa_ref, pb_ref, e_ref, o_ref):
    o_ref[...] = (pa_ref[0] + pa_ref[1]) + (pb_ref[0] + pb_ref[1]) + e_ref[...]


def _combine(pa, pb, energy_pad):
    return pl.pallas_call(
        _comb_body,
        in_specs=[
            pl.BlockSpec((NC, MR, 128), lambda: (0, 0, 0)),
            pl.BlockSpec((NC, MR, 128), lambda: (0, 0, 0)),
            pl.BlockSpec((MR, 128), lambda: (0, 0)),
        ],
        out_specs=pl.BlockSpec((MR, 128), lambda: (0, 0)),
        out_shape=jax.ShapeDtypeStruct((MR, 128), jnp.float32),
    )(pa, pb, energy_pad)


def kernel(_T0, _idx_m, _energy, W, b):
    idx32 = _idx_m.astype(jnp.int32)
    w = W.reshape(D)
    eye = jnp.eye(FOLD, dtype=jnp.float32)
    g = (eye[:, None, :] * w[None, :, None]).reshape(KW, FOLD)
    y_a = _matvec(_T0, g, b, SPLIT, 0, 2000).reshape(SPLIT)
    y_b = _matvec(_T0, g, b, N - SPLIT, SPLIT, 2000).reshape(N - SPLIT)
    pa = _segsum_a(y_a, idx32)
    pb = _segsum_b(y_b, idx32)
    energy_pad = jnp.pad(_energy, (0, MP - M)).reshape(MR, 128)
    out = _combine(pa.reshape(NC, MR, 128), pb.reshape(NC, MR, 128),
                   energy_pad)
    return out.reshape(MP)[:M]


# speedup vs baseline: 1.0020x; 1.0020x over previous
import functools

import jax
import jax.numpy as jnp
from jax import lax
from jax.experimental import pallas as pl
from jax.experimental.pallas import tpu as pltpu
from jax.experimental.pallas import tpu_sc as plsc

N = 320000
D = 128
M = 10000


FOLD = 16
KW = D * FOLD
SPLIT = 128000


def _matvec(t0, g, b, n_atoms, off_atoms, br):
    blk = br * FOLD
    ng = n_atoms // blk
    offb = off_atoms // blk

    def _mv_body(t_ref, g_ref, b_ref, o_ref):
        t = t_ref[...].reshape(br, KW)
        y = lax.dot_general(t, g_ref[...], (((1,), (0,)), ((), ())),
                            preferred_element_type=jnp.float32)
        o_ref[...] = y + b_ref[0]

    return pl.pallas_call(
        _mv_body,
        grid=(ng,),
        in_specs=[
            pl.BlockSpec((blk, D), lambda i: (i + offb, 0)),
            pl.BlockSpec((KW, FOLD), lambda i: (0, 0)),
            pl.BlockSpec(memory_space=pltpu.SMEM),
        ],
        out_specs=pl.BlockSpec((br, FOLD), lambda i: (i, 0)),
        out_shape=jax.ShapeDtypeStruct((n_atoms // FOLD, FOLD), jnp.float32),
    )(t0, g, b)


NC = 2
NS = 16
NW = NC * NS
MP = 10240
RED = MP // NS


def _gather16(x, idx):
    return lax.gather(
        x, idx[:, None],
        dimension_numbers=lax.GatherDimensionNumbers(
            offset_dims=(), collapsed_slice_dims=(0,), start_index_map=(0,)),
        slice_sizes=(1,),
        mode=lax.GatherScatterMode.PROMISE_IN_BOUNDS)


def _make_sc_body(chunk, idx_off):
    nvec = chunk // 16

    def _sc_body(y_hbm, idx_hbm, out_hbm, yv, iv, tbl, red, acc, shared):
        c = lax.axis_index("c")
        s = lax.axis_index("s")
        wid = c * NS + s
        base = wid * chunk
        pltpu.sync_copy(y_hbm.at[pl.ds(base, chunk)], yv)
        pltpu.sync_copy(idx_hbm.at[pl.ds(idx_off + base, chunk)], iv)

        zero = jnp.zeros((16,), jnp.float32)
        iota = lax.iota(jnp.int32, 16)
        last = iota == 15
        notlast = jnp.logical_not(last)

        def zbody(i, _):
            tbl[pl.ds(i * 16, 16)] = zero
            return 0

        lax.fori_loop(0, MP // 16, zbody, 0, unroll=8)

        def scat(yk, ik, ink):
            sk = plsc.cumsum(yk)
            e = ik != ink
            plsc.addupdate_scatter(tbl, [ik], sk,
                                   mask=jnp.logical_or(e, last))
            plsc.addupdate_scatter(tbl, [ink], -sk,
                                   mask=jnp.logical_and(e, notlast))

        def body(k, _):
            off = k * 16
            scat(yv[pl.ds(off, 16)], iv[pl.ds(off, 16)],
                 iv[pl.ds(off + 1, 16)])
            return 0

        lax.fori_loop(0, nvec - 1, body, 0, unroll=5)

        off = (nvec - 1) * 16
        ik = iv[pl.ds(off, 16)]
        scat(yv[pl.ds(off, 16)], ik,
             _gather16(ik, jnp.minimum(iota + 1, 15)))

        pltpu.sync_copy(tbl, shared.at[s])
        plsc.subcore_barrier()

        rbase = s * RED
        for t in range(NS):
            pltpu.sync_copy(shared.at[t, pl.ds(rbase, RED)], red.at[t])

        def rbody(i, _):
            off = i * 16
            v = red[0, pl.ds(off, 16)]
            for t in range(1, NS):
                v = v + red[t, pl.ds(off, 16)]
            acc[pl.ds(off, 16)] = v
            return 0

        lax.fori_loop(0, RED // 16, rbody, 0, unroll=2)
        pltpu.sync_copy(acc, out_hbm.at[c, pl.ds(rbase, RED)])

    return _sc_body


def _make_segsum(n_atoms, idx_off):
    chunk = n_atoms // NW
    return functools.partial(
        pl.kernel,
        mesh=plsc.VectorSubcoreMesh(core_axis_name="c", subcore_axis_name="s"),
        out_type=jax.ShapeDtypeStruct((NC, MP), jnp.float32),
        compiler_params=pltpu.CompilerParams(needs_layout_passes=False),
        scratch_types=[
            pltpu.VMEM((chunk,), jnp.float32),
            pltpu.VMEM((chunk,), jnp.int32),
            pltpu.VMEM((MP,), jnp.float32),
            pltpu.VMEM((NS, RED), jnp.float32),
            pltpu.VMEM((RED,), jnp.float32),
            pltpu.VMEM_SHARED((NS, MP), jnp.float32),
        ],
    )(_make_sc_body(chunk, idx_off))


_segsum_a = _make_segsum(SPLIT, 0)
_segsum_b = _make_segsum(N - SPLIT, SPLIT)


MR = MP // 128


def _comb_body(pa_ref, pb_ref, e_ref, o_ref):
    o_ref[...] = (pa_ref[0] + pa_ref[1]) + (pb_ref[0] + pb_ref[1]) + e_ref[...]


def _combine(pa, pb, energy_pad):
    return pl.pallas_call(
        _comb_body,
        in_specs=[
            pl.BlockSpec((NC, MR, 128), lambda: (0, 0, 0)),
            pl.BlockSpec((NC, MR, 128), lambda: (0, 0, 0)),
            pl.BlockSpec((MR, 128), lambda: (0, 0)),
        ],
        out_specs=pl.BlockSpec((MR, 128), lambda: (0, 0)),
        out_shape=jax.ShapeDtypeStruct((MR, 128), jnp.float32),
    )(pa, pb, energy_pad)


def kernel(_T0, _idx_m, _energy, W, b):
    idx32 = _idx_m.astype(jnp.int32)
    w = W.reshape(D)
    eye = jnp.eye(FOLD, dtype=jnp.float32)
    g = (eye[:, None, :] * w[None, :, None]).reshape(KW, FOLD)
    y_a = _matvec(_T0, g, b, SPLIT, 0, 1000).reshape(SPLIT)
    y_b = _matvec(_T0, g, b, N - SPLIT, SPLIT, 1000).reshape(N - SPLIT)
    pa = _segsum_a(y_a, idx32)
    pb = _segsum_b(y_b, idx32)
    energy_pad = jnp.pad(_energy, (0, MP - M)).reshape(MR, 128)
    out = _combine(pa.reshape(NC, MR, 128), pb.reshape(NC, MR, 128),
                   energy_pad)
    return out.reshape(MP)[:M]


# speedup vs baseline: 1.0099x; 1.0078x over previous
import functools

import jax
import jax.numpy as jnp
from jax import lax
from jax.experimental import pallas as pl
from jax.experimental.pallas import tpu as pltpu
from jax.experimental.pallas import tpu_sc as plsc

N = 320000
D = 128
M = 10000


FOLD = 32
KW = D * FOLD
SPLIT = 128000


def _matvec(t0, g, b, n_atoms, off_atoms, br):
    blk = br * FOLD
    ng = n_atoms // blk
    offb = off_atoms // blk

    def _mv_body(t_ref, g_ref, b_ref, o_ref):
        t = t_ref[...].reshape(br, KW)
        y = lax.dot_general(t, g_ref[...], (((1,), (0,)), ((), ())),
                            preferred_element_type=jnp.float32)
        o_ref[...] = y + b_ref[0]

    return pl.pallas_call(
        _mv_body,
        grid=(ng,),
        in_specs=[
            pl.BlockSpec((blk, D), lambda i: (i + offb, 0)),
            pl.BlockSpec((KW, FOLD), lambda i: (0, 0)),
            pl.BlockSpec(memory_space=pltpu.SMEM),
        ],
        out_specs=pl.BlockSpec((br, FOLD), lambda i: (i, 0)),
        out_shape=jax.ShapeDtypeStruct((n_atoms // FOLD, FOLD), jnp.float32),
    )(t0, g, b)


NC = 2
NS = 16
NW = NC * NS
MP = 10240
RED = MP // NS


def _gather16(x, idx):
    return lax.gather(
        x, idx[:, None],
        dimension_numbers=lax.GatherDimensionNumbers(
            offset_dims=(), collapsed_slice_dims=(0,), start_index_map=(0,)),
        slice_sizes=(1,),
        mode=lax.GatherScatterMode.PROMISE_IN_BOUNDS)


def _make_sc_body(chunk, idx_off):
    nvec = chunk // 16

    def _sc_body(y_hbm, idx_hbm, out_hbm, yv, iv, tbl, red, acc, shared):
        c = lax.axis_index("c")
        s = lax.axis_index("s")
        wid = c * NS + s
        base = wid * chunk
        pltpu.sync_copy(y_hbm.at[pl.ds(base, chunk)], yv)
        pltpu.sync_copy(idx_hbm.at[pl.ds(idx_off + base, chunk)], iv)

        zero = jnp.zeros((16,), jnp.float32)
        iota = lax.iota(jnp.int32, 16)
        last = iota == 15
        notlast = jnp.logical_not(last)

        def zbody(i, _):
            tbl[pl.ds(i * 16, 16)] = zero
            return 0

        lax.fori_loop(0, MP // 16, zbody, 0, unroll=8)

        def scat(yk, ik, ink):
            sk = plsc.cumsum(yk)
            e = ik != ink
            plsc.addupdate_scatter(tbl, [ik], sk,
                                   mask=jnp.logical_or(e, last))
            plsc.addupdate_scatter(tbl, [ink], -sk,
                                   mask=jnp.logical_and(e, notlast))

        def body(k, _):
            off = k * 16
            scat(yv[pl.ds(off, 16)], iv[pl.ds(off, 16)],
                 iv[pl.ds(off + 1, 16)])
            return 0

        lax.fori_loop(0, nvec - 1, body, 0, unroll=5)

        off = (nvec - 1) * 16
        ik = iv[pl.ds(off, 16)]
        scat(yv[pl.ds(off, 16)], ik,
             _gather16(ik, jnp.minimum(iota + 1, 15)))

        pltpu.sync_copy(tbl, shared.at[s])
        plsc.subcore_barrier()

        rbase = s * RED
        for t in range(NS):
            pltpu.sync_copy(shared.at[t, pl.ds(rbase, RED)], red.at[t])

        def rbody(i, _):
            off = i * 16
            v = red[0, pl.ds(off, 16)]
            for t in range(1, NS):
                v = v + red[t, pl.ds(off, 16)]
            acc[pl.ds(off, 16)] = v
            return 0

        lax.fori_loop(0, RED // 16, rbody, 0, unroll=2)
        pltpu.sync_copy(acc, out_hbm.at[c, pl.ds(rbase, RED)])

    return _sc_body


def _make_segsum(n_atoms, idx_off):
    chunk = n_atoms // NW
    return functools.partial(
        pl.kernel,
        mesh=plsc.VectorSubcoreMesh(core_axis_name="c", subcore_axis_name="s"),
        out_type=jax.ShapeDtypeStruct((NC, MP), jnp.float32),
        compiler_params=pltpu.CompilerParams(needs_layout_passes=False),
        scratch_types=[
            pltpu.VMEM((chunk,), jnp.float32),
            pltpu.VMEM((chunk,), jnp.int32),
            pltpu.VMEM((MP,), jnp.float32),
            pltpu.VMEM((NS, RED), jnp.float32),
            pltpu.VMEM((RED,), jnp.float32),
            pltpu.VMEM_SHARED((NS, MP), jnp.float32),
        ],
    )(_make_sc_body(chunk, idx_off))


_segsum_a = _make_segsum(SPLIT, 0)
_segsum_b = _make_segsum(N - SPLIT, SPLIT)


MR = MP // 128


def _comb_body(pa_ref, pb_ref, e_ref, o_ref):
    o_ref[...] = (pa_ref[0] + pa_ref[1]) + (pb_ref[0] + pb_ref[1]) + e_ref[...]


def _combine(pa, pb, energy_pad):
    return pl.pallas_call(
        _comb_body,
        in_specs=[
            pl.BlockSpec((NC, MR, 128), lambda: (0, 0, 0)),
            pl.BlockSpec((NC, MR, 128), lambda: (0, 0, 0)),
            pl.BlockSpec((MR, 128), lambda: (0, 0)),
        ],
        out_specs=pl.BlockSpec((MR, 128), lambda: (0, 0)),
        out_shape=jax.ShapeDtypeStruct((MR, 128), jnp.float32),
    )(pa, pb, energy_pad)


def kernel(_T0, _idx_m, _energy, W, b):
    idx32 = _idx_m.astype(jnp.int32)
    w = W.reshape(D)
    eye = jnp.eye(FOLD, dtype=jnp.float32)
    g = (eye[:, None, :] * w[None, :, None]).reshape(KW, FOLD)
    y_a = _matvec(_T0, g, b, SPLIT, 0, 1000).reshape(SPLIT)
    y_b = _matvec(_T0, g, b, N - SPLIT, SPLIT, 1000).reshape(N - SPLIT)
    pa = _segsum_a(y_a, idx32)
    pb = _segsum_b(y_b, idx32)
    energy_pad = jnp.pad(_energy, (0, MP - M)).reshape(MR, 128)
    out = _combine(pa.reshape(NC, MR, 128), pb.reshape(NC, MR, 128),
                   energy_pad)
    return out.reshape(MP)[:M]
